# baseline (speedup 1.0000x reference)
import jax
import jax.numpy as jnp
from jax.experimental import pallas as pl
from jax.experimental.pallas import tpu as pltpu

_VOCAB = 100000
_BATCH = 1024
_VT = 2048
_NV = (_VOCAB + _VT - 1) // _VT


_BP = 32


def _body(emb_ref, out_ref):
    out_ref[...] = jnp.broadcast_to(emb_ref[0, 0], (_BP, _VOCAB))


def kernel(inputs, table, W, b):
    return pl.pallas_call(
        _body,
        grid=(_BATCH // _BP,),
        in_specs=[pl.BlockSpec((8, 64), lambda j: (0, 0))],
        out_specs=pl.BlockSpec((_BP, _VOCAB), lambda j: (j, 0)),
        out_shape=jax.ShapeDtypeStruct((_BATCH, _VOCAB), jnp.float32),
    )(table)


# speedup vs baseline: 1.5229x; 1.5229x over previous
import jax
import jax.numpy as jnp
from jax.experimental import pallas as pl
from jax.experimental.pallas import tpu as pltpu
from jax.experimental.pallas import tpu_sc as plsc

_VOCAB = 100000
_EMBED = 64
_BATCH = 1024
_CTX = 20

_NW = 32
_BPW = (_BATCH * _CTX) // _NW

_VT = 2000
_NV = _VOCAB // _VT
_NSPLIT = 5
_RCHUNK = _VT // _NSPLIT


def _sc_gather(table, flat_idx):
    n = _BATCH * _CTX
    mesh = plsc.VectorSubcoreMesh(core_axis_name="c", subcore_axis_name="s")

    @pl.kernel(out_type=jax.ShapeDtypeStruct((n, _EMBED), table.dtype),
               mesh=mesh,
               compiler_params=pltpu.CompilerParams(use_tc_tiling_on_sc=False),
               scratch_types=[
                   pltpu.VMEM((_BPW,), jnp.int32),
                   pltpu.VMEM((_BPW, _EMBED), jnp.float32),
                   pltpu.SemaphoreType.DMA,
               ])
    def gather_kernel(table_hbm, idx_hbm, out_hbm, idx_v, rows_v, sem):
        wid = jax.lax.axis_index("s") * 2 + jax.lax.axis_index("c")
        base = wid * _BPW
        pltpu.sync_copy(idx_hbm.at[pl.ds(base, _BPW)], idx_v)
        pltpu.async_copy(table_hbm.at[idx_v], rows_v, sem).wait()
        pltpu.sync_copy(rows_v, out_hbm.at[pl.ds(base, _BPW)])

    return gather_kernel(table, flat_idx)


def _mean_body(emb_full_ref, embt_ref):
    acc = emb_full_ref[pl.ds(0, _BATCH), :]
    for c in range(1, _CTX):
        acc = acc + emb_full_ref[pl.ds(c * _BATCH, _BATCH), :]
    emb = (acc * (1.0 / _CTX)).astype(jnp.bfloat16)
    embt_ref[...] = emb.T


def _mean_t(emb_full):
    return pl.pallas_call(
        _mean_body,
        out_shape=jax.ShapeDtypeStruct((_EMBED, _BATCH), jnp.bfloat16),
    )(emb_full)


def _project_body(embt_ref, w_ref, b_ref, out_hbm, out_buf, sems):
    j = pl.program_id(0)
    cur = jax.lax.rem(j, 2)
    base = j * _VT

    @pl.when(j >= 2)
    def _():
        for k in range(_NSPLIT):
            pltpu.make_async_copy(
                out_buf.at[cur, pl.ds(k * _RCHUNK, _RCHUNK), :],
                out_hbm.at[pl.ds(k * _RCHUNK, _RCHUNK), :],
                sems.at[cur, k],
            ).wait()

    out_buf[cur] = jax.lax.dot_general(
        w_ref[...].astype(jnp.bfloat16), embt_ref[...],
        dimension_numbers=(((1,), (0,)), ((), ())),
        preferred_element_type=jnp.float32,
    ) + b_ref[...]

    for k in range(_NSPLIT):
        pltpu.make_async_copy(
            out_buf.at[cur, pl.ds(k * _RCHUNK, _RCHUNK), :],
            out_hbm.at[pl.ds(base + k * _RCHUNK, _RCHUNK), :],
            sems.at[cur, k],
        ).start()

    @pl.when(j == _NV - 1)
    def _():
        prev = jax.lax.rem(j + 1, 2)
        for k in range(_NSPLIT):
            pltpu.make_async_copy(
                out_buf.at[prev, pl.ds(k * _RCHUNK, _RCHUNK), :],
                out_hbm.at[pl.ds(k * _RCHUNK, _RCHUNK), :],
                sems.at[prev, k],
            ).wait()
        for k in range(_NSPLIT):
            pltpu.make_async_copy(
                out_buf.at[cur, pl.ds(k * _RCHUNK, _RCHUNK), :],
                out_hbm.at[pl.ds(k * _RCHUNK, _RCHUNK), :],
                sems.at[cur, k],
            ).wait()


def _project_t(embt, W, b2):
    return pl.pallas_call(
        _project_body,
        grid=(_NV,),
        in_specs=[
            pl.BlockSpec((_EMBED, _BATCH), lambda j: (0, 0)),
            pl.BlockSpec((_VT, _EMBED), lambda j: (j, 0)),
            pl.BlockSpec((_VT, 1), lambda j: (j, 0)),
        ],
        out_specs=pl.BlockSpec(memory_space=pl.ANY),
        out_shape=jax.ShapeDtypeStruct((_VOCAB, _BATCH), jnp.float32),
        scratch_shapes=[
            pltpu.VMEM((2, _VT, _BATCH), jnp.float32),
            pltpu.SemaphoreType.DMA((2, _NSPLIT)),
        ],
        compiler_params=pltpu.CompilerParams(
            dimension_semantics=("arbitrary",)),
    )(embt, W, b2)


def kernel(inputs, table, W, b):
    flat_idx = inputs.T.reshape(_BATCH * _CTX).astype(jnp.int32)
    emb_full = _sc_gather(table, flat_idx)
    embt = _mean_t(emb_full)
    logits_t = _project_t(embt, W, b.reshape(_VOCAB, 1))
    return logits_t.T


# speedup vs baseline: 1.9388x; 1.2731x over previous
import jax
import jax.numpy as jnp
from jax.experimental import pallas as pl
from jax.experimental.pallas import tpu as pltpu
from jax.experimental.pallas import tpu_sc as plsc

_VOCAB = 100000
_EMBED = 64
_BATCH = 1024
_CTX = 20

_NW = 32
_IPW = _BATCH // _NW
_RPAD = 24
_NLANE = 16

_VT = 2000
_NV = _VOCAB // _VT
_NSPLIT = 10
_RCHUNK = _VT // _NSPLIT


def _sc_gather_mean(table, inputs):
    mesh = plsc.VectorSubcoreMesh(core_axis_name="c", subcore_axis_name="s")

    @pl.kernel(out_type=jax.ShapeDtypeStruct((_BATCH, _EMBED), table.dtype),
               mesh=mesh,
               compiler_params=pltpu.CompilerParams(use_tc_tiling_on_sc=False),
               scratch_types=[
                   pltpu.VMEM((_IPW, _CTX), jnp.int32),
                   pltpu.VMEM((_IPW * _RPAD, _EMBED), jnp.float32),
                   pltpu.VMEM((_IPW, _EMBED), jnp.float32),
                   pltpu.SemaphoreType.DMA,
               ])
    def gather_kernel(table_hbm, idx_hbm, out_hbm, idx_v, rows_v, emb_v, sem):
        wid = jax.lax.axis_index("s") * 2 + jax.lax.axis_index("c")
        base = wid * _IPW
        pltpu.sync_copy(idx_hbm.at[pl.ds(base, _IPW), :], idx_v)
        copies = []
        for i in range(_IPW):
            copies.append(pltpu.async_copy(
                table_hbm.at[idx_v.at[i]],
                rows_v.at[pl.ds(i * _RPAD, _CTX)],
                sem,
            ))
        for cp in copies:
            cp.wait()

        @pl.loop(0, _IPW)
        def _(i):
            for l in range(_EMBED // _NLANE):
                sl = pl.ds(l * _NLANE, _NLANE)
                acc = rows_v[i * _RPAD, sl]
                for c in range(1, _CTX):
                    acc = acc + rows_v[i * _RPAD + c, sl]
                emb_v[i, sl] = acc * (1.0 / _CTX)

        pltpu.sync_copy(emb_v, out_hbm.at[pl.ds(base, _IPW)])

    return gather_kernel(table, inputs)


def _project_body(emb_ref, w_ref, b_ref, out_hbm, out_buf, embt, sems):
    j = pl.program_id(0)
    cur = jax.lax.rem(j, 2)
    base = j * _VT

    @pl.when(j == 0)
    def _():
        embt[...] = emb_ref[...].astype(jnp.bfloat16).T

    @pl.when(j >= 2)
    def _():
        for k in range(_NSPLIT):
            pltpu.make_async_copy(
                out_buf.at[cur, pl.ds(k * _RCHUNK, _RCHUNK), :],
                out_hbm.at[pl.ds(k * _RCHUNK, _RCHUNK), :],
                sems.at[cur, k],
            ).wait()

    out_buf[cur] = jax.lax.dot_general(
        w_ref[...].astype(jnp.bfloat16), embt[...],
        dimension_numbers=(((1,), (0,)), ((), ())),
        preferred_element_type=jnp.float32,
    ) + b_ref[...].reshape(1, _VT).T

    for k in range(_NSPLIT):
        pltpu.make_async_copy(
            out_buf.at[cur, pl.ds(k * _RCHUNK, _RCHUNK), :],
            out_hbm.at[pl.ds(base + k * _RCHUNK, _RCHUNK), :],
            sems.at[cur, k],
        ).start()

    @pl.when(j == _NV - 1)
    def _():
        prev = jax.lax.rem(j + 1, 2)
        for k in range(_NSPLIT):
            pltpu.make_async_copy(
                out_buf.at[prev, pl.ds(k * _RCHUNK, _RCHUNK), :],
                out_hbm.at[pl.ds(k * _RCHUNK, _RCHUNK), :],
                sems.at[prev, k],
            ).wait()
        for k in range(_NSPLIT):
            pltpu.make_async_copy(
                out_buf.at[cur, pl.ds(k * _RCHUNK, _RCHUNK), :],
                out_hbm.at[pl.ds(k * _RCHUNK, _RCHUNK), :],
                sems.at[cur, k],
            ).wait()


def _project_t(emb, W, b2):
    return pl.pallas_call(
        _project_body,
        grid=(_NV,),
        in_specs=[
            pl.BlockSpec((_BATCH, _EMBED), lambda j: (0, 0)),
            pl.BlockSpec((_VT, _EMBED), lambda j: (j, 0)),
            pl.BlockSpec((1, 1, _VT), lambda j: (j, 0, 0)),
        ],
        out_specs=pl.BlockSpec(memory_space=pl.ANY),
        out_shape=jax.ShapeDtypeStruct((_VOCAB, _BATCH), jnp.float32),
        scratch_shapes=[
            pltpu.VMEM((2, _VT, _BATCH), jnp.float32),
            pltpu.VMEM((_EMBED, _BATCH), jnp.bfloat16),
            pltpu.SemaphoreType.DMA((2, _NSPLIT)),
        ],
        compiler_params=pltpu.CompilerParams(
            dimension_semantics=("arbitrary",)),
    )(emb, W, b2)


def kernel(inputs, table, W, b):
    emb = _sc_gather_mean(table, inputs.astype(jnp.int32))
    logits_t = _project_t(emb, W, b.reshape(_NV, 1, _VT))
    return logits_t.T


# speedup vs baseline: 2.2749x; 1.1733x over previous
import jax
import jax.numpy as jnp
from jax.experimental import pallas as pl
from jax.experimental.pallas import tpu as pltpu
from jax.experimental.pallas import tpu_sc as plsc

_VOCAB = 100000
_EMBED = 64
_BATCH = 1024
_CTX = 20

_NW = 32
_IPW = _BATCH // _NW
_RPAD = 24
_NLANE = 16

_VT = 2048
_NV = (_VOCAB + _VT - 1) // _VT
_VTAIL = _VOCAB - (_NV - 1) * _VT
_NSPLIT = 8
_RCHUNK = _VT // _NSPLIT
_TSPLIT = 4
_TCHUNK = _VTAIL // _TSPLIT


def _sc_gather_mean(table, inputs):
    mesh = plsc.VectorSubcoreMesh(core_axis_name="c", subcore_axis_name="s")

    @pl.kernel(out_type=jax.ShapeDtypeStruct((_BATCH, _EMBED), table.dtype),
               mesh=mesh,
               compiler_params=pltpu.CompilerParams(use_tc_tiling_on_sc=False),
               scratch_types=[
                   pltpu.VMEM((_IPW, _CTX), jnp.int32),
                   pltpu.VMEM((_IPW * _RPAD, _EMBED), jnp.float32),
                   pltpu.VMEM((_IPW, _EMBED), jnp.float32),
                   pltpu.SemaphoreType.DMA,
               ])
    def gather_kernel(table_hbm, idx_hbm, out_hbm, idx_v, rows_v, emb_v, sem):
        wid = jax.lax.axis_index("s") * 2 + jax.lax.axis_index("c")
        base = wid * _IPW
        pltpu.sync_copy(idx_hbm.at[pl.ds(base, _IPW), :], idx_v)
        copies = []
        for i in range(_IPW):
            copies.append(pltpu.async_copy(
                table_hbm.at[idx_v.at[i]],
                rows_v.at[pl.ds(i * _RPAD, _CTX)],
                sem,
            ))
        for cp in copies:
            cp.wait()

        @pl.loop(0, _IPW)
        def _(i):
            for l in range(_EMBED // _NLANE):
                sl = pl.ds(l * _NLANE, _NLANE)
                acc = rows_v[i * _RPAD, sl]
                for c in range(1, _CTX):
                    acc = acc + rows_v[i * _RPAD + c, sl]
                emb_v[i, sl] = acc * (1.0 / _CTX)

        pltpu.sync_copy(emb_v, out_hbm.at[pl.ds(base, _IPW)])

    return gather_kernel(table, inputs)


def _project_body(emb_ref, wt_ref, b_ref, out_hbm, out_buf, embt, sems):
    j = pl.program_id(0)
    cur = jax.lax.rem(j, 2)
    base = j * _VT

    @pl.when(j == 0)
    def _():
        embt[...] = emb_ref[...].astype(jnp.bfloat16).T

    @pl.when(j >= 2)
    def _():
        for k in range(_NSPLIT):
            pltpu.make_async_copy(
                out_buf.at[cur, pl.ds(k * _RCHUNK, _RCHUNK), :],
                out_hbm.at[pl.ds(k * _RCHUNK, _RCHUNK), :],
                sems.at[cur, k],
            ).wait()

    out_buf[cur] = jax.lax.dot_general(
        wt_ref[...].astype(jnp.bfloat16), embt[...],
        dimension_numbers=(((0,), (0,)), ((), ())),
        preferred_element_type=jnp.float32,
    ) + b_ref[...].reshape(1, _VT).T

    @pl.when(j < _NV - 1)
    def _():
        for k in range(_NSPLIT):
            pltpu.make_async_copy(
                out_buf.at[cur, pl.ds(k * _RCHUNK, _RCHUNK), :],
                out_hbm.at[pl.ds(base + k * _RCHUNK, _RCHUNK), :],
                sems.at[cur, k],
            ).start()

    @pl.when(j == _NV - 1)
    def _():
        for k in range(_TSPLIT):
            pltpu.make_async_copy(
                out_buf.at[cur, pl.ds(k * _TCHUNK, _TCHUNK), :],
                out_hbm.at[pl.ds(base + k * _TCHUNK, _TCHUNK), :],
                sems.at[cur, k],
            ).start()
        prev = jax.lax.rem(j + 1, 2)
        for k in range(_NSPLIT):
            pltpu.make_async_copy(
                out_buf.at[prev, pl.ds(k * _RCHUNK, _RCHUNK), :],
                out_hbm.at[pl.ds(k * _RCHUNK, _RCHUNK), :],
                sems.at[prev, k],
            ).wait()
        for k in range(_TSPLIT):
            pltpu.make_async_copy(
                out_buf.at[cur, pl.ds(k * _TCHUNK, _TCHUNK), :],
                out_hbm.at[pl.ds(k * _TCHUNK, _TCHUNK), :],
                sems.at[cur, k],
            ).wait()


def _project_t(emb, WT, b2):
    return pl.pallas_call(
        _project_body,
        grid=(_NV,),
        in_specs=[
            pl.BlockSpec((_BATCH, _EMBED), lambda j: (0, 0)),
            pl.BlockSpec((_EMBED, _VT), lambda j: (0, j)),
            pl.BlockSpec((1, 1, _VT), lambda j: (j, 0, 0)),
        ],
        out_specs=pl.BlockSpec(memory_space=pl.ANY),
        out_shape=jax.ShapeDtypeStruct((_VOCAB, _BATCH), jnp.float32),
        scratch_shapes=[
            pltpu.VMEM((2, _VT, _BATCH), jnp.float32),
            pltpu.VMEM((_EMBED, _BATCH), jnp.bfloat16),
            pltpu.SemaphoreType.DMA((2, _NSPLIT)),
        ],
        compiler_params=pltpu.CompilerParams(
            dimension_semantics=("arbitrary",)),
    )(emb, WT, b2)


def kernel(inputs, table, W, b):
    emb = _sc_gather_mean(table, inputs.astype(jnp.int32))
    b_pad = jnp.pad(b, (0, _NV * _VT - _VOCAB)).reshape(_NV, 1, _VT)
    logits_t = _project_t(emb, W.T, b_pad)
    return logits_t.T


# speedup vs baseline: 2.2847x; 1.0043x over previous
import jax
import jax.numpy as jnp
from jax.experimental import pallas as pl
from jax.experimental.pallas import tpu as pltpu
from jax.experimental.pallas import tpu_sc as plsc

_VOCAB = 100000
_EMBED = 64
_BATCH = 1024
_CTX = 20

_NW = 32
_IPW = _BATCH // _NW
_RPAD = 24
_NLANE = 16

_VT = 2048
_NV = (_VOCAB + _VT - 1) // _VT
_VTAIL = _VOCAB - (_NV - 1) * _VT
_NSPLIT = 8
_RCHUNK = _VT // _NSPLIT
_TSPLIT = 4
_TCHUNK = _VTAIL // _TSPLIT


def _sc_gather_mean(table, inputs):
    mesh = plsc.VectorSubcoreMesh(core_axis_name="c", subcore_axis_name="s")

    @pl.kernel(out_type=jax.ShapeDtypeStruct((_BATCH, _EMBED), table.dtype),
               mesh=mesh,
               compiler_params=pltpu.CompilerParams(use_tc_tiling_on_sc=False),
               scratch_types=[
                   pltpu.VMEM((_IPW, _CTX), jnp.int32),
                   pltpu.VMEM((_IPW * _RPAD, _EMBED), jnp.float32),
                   pltpu.VMEM((_IPW, _EMBED), jnp.float32),
                   pltpu.SemaphoreType.DMA,
               ])
    def gather_kernel(table_hbm, idx_hbm, out_hbm, idx_v, rows_v, emb_v, sem):
        wid = jax.lax.axis_index("s") * 2 + jax.lax.axis_index("c")
        base = wid * _IPW
        pltpu.sync_copy(idx_hbm.at[pl.ds(base, _IPW), :], idx_v)
        copies = []
        for i in range(_IPW):
            copies.append(pltpu.async_copy(
                table_hbm.at[idx_v.at[i]],
                rows_v.at[pl.ds(i * _RPAD, _CTX)],
                sem,
            ))
        for cp in copies:
            cp.wait()

        @pl.loop(0, _IPW)
        def _(i):
            for l in range(_EMBED // _NLANE):
                sl = pl.ds(l * _NLANE, _NLANE)
                acc = rows_v[i * _RPAD, sl]
                for c in range(1, _CTX):
                    acc = acc + rows_v[i * _RPAD + c, sl]
                emb_v[i, sl] = acc * (1.0 / _CTX)

        pltpu.sync_copy(emb_v, out_hbm.at[pl.ds(base, _IPW)])

    return gather_kernel(table, inputs)


_NBUF = 3


def _project_body(emb_ref, wt_ref, b_ref, out_hbm, out_buf, embt, sems):
    j = pl.program_id(0)
    cur = jax.lax.rem(j, _NBUF)
    base = j * _VT

    @pl.when(j == 0)
    def _():
        embt[...] = emb_ref[...].astype(jnp.bfloat16).T

    @pl.when(j >= _NBUF)
    def _():
        for k in range(_NSPLIT):
            pltpu.make_async_copy(
                out_buf.at[cur, pl.ds(k * _RCHUNK, _RCHUNK), :],
                out_hbm.at[pl.ds(k * _RCHUNK, _RCHUNK), :],
                sems.at[cur, k],
            ).wait()

    out_buf[cur] = jax.lax.dot_general(
        wt_ref[...].astype(jnp.bfloat16), embt[...],
        dimension_numbers=(((0,), (0,)), ((), ())),
        preferred_element_type=jnp.float32,
    ) + b_ref[...].reshape(1, _VT).T

    @pl.when(j < _NV - 1)
    def _():
        for k in range(_NSPLIT):
            pltpu.make_async_copy(
                out_buf.at[cur, pl.ds(k * _RCHUNK, _RCHUNK), :],
                out_hbm.at[pl.ds(base + k * _RCHUNK, _RCHUNK), :],
                sems.at[cur, k],
            ).start()

    @pl.when(j == _NV - 1)
    def _():
        for k in range(_TSPLIT):
            pltpu.make_async_copy(
                out_buf.at[cur, pl.ds(k * _TCHUNK, _TCHUNK), :],
                out_hbm.at[pl.ds(base + k * _TCHUNK, _TCHUNK), :],
                sems.at[cur, k],
            ).start()
        for d in range(1, _NBUF):
            prev = jax.lax.rem(j + _NBUF - d, _NBUF)
            for k in range(_NSPLIT):
                pltpu.make_async_copy(
                    out_buf.at[prev, pl.ds(k * _RCHUNK, _RCHUNK), :],
                    out_hbm.at[pl.ds(k * _RCHUNK, _RCHUNK), :],
                    sems.at[prev, k],
                ).wait()
        for k in range(_TSPLIT):
            pltpu.make_async_copy(
                out_buf.at[cur, pl.ds(k * _TCHUNK, _TCHUNK), :],
                out_hbm.at[pl.ds(k * _TCHUNK, _TCHUNK), :],
                sems.at[cur, k],
            ).wait()


def _project_t(emb, WT, b2):
    return pl.pallas_call(
        _project_body,
        grid=(_NV,),
        in_specs=[
            pl.BlockSpec((_BATCH, _EMBED), lambda j: (0, 0)),
            pl.BlockSpec((_EMBED, _VT), lambda j: (0, j)),
            pl.BlockSpec((1, 1, _VT), lambda j: (j, 0, 0)),
        ],
        out_specs=pl.BlockSpec(memory_space=pl.ANY),
        out_shape=jax.ShapeDtypeStruct((_VOCAB, _BATCH), jnp.float32),
        scratch_shapes=[
            pltpu.VMEM((_NBUF, _VT, _BATCH), jnp.float32),
            pltpu.VMEM((_EMBED, _BATCH), jnp.bfloat16),
            pltpu.SemaphoreType.DMA((_NBUF, _NSPLIT)),
        ],
        compiler_params=pltpu.CompilerParams(
            dimension_semantics=("arbitrary",)),
    )(emb, WT, b2)


def kernel(inputs, table, W, b):
    emb = _sc_gather_mean(table, inputs.astype(jnp.int32))
    b_pad = jnp.pad(b, (0, _NV * _VT - _VOCAB)).reshape(_NV, 1, _VT)
    logits_t = _project_t(emb, W.T, b_pad)
    return logits_t.T
